# baseline (speedup 1.0000x reference)
import jax
import jax.numpy as jnp
from jax import lax
from jax.experimental import pallas as pl
from jax.experimental.pallas import tpu as pltpu
from jax.experimental.pallas import tpu_sc as plsc

_NUM_EMBEDDINGS = 8192
_EMBEDDING_DIM = 64
_BETA = 0.25
_B = 16384

_NC = 2
_NS = 16
_NW = _NC * _NS
_LANES = 16
_BPW = _B // _NW

_BM = 2048
_BK = 4096


def _argmin_body(z2_ref, c2_ref, z_ref, c_ref, idx_ref,
                 run_min, run_idx, m0_ref, i0_ref):
    j = pl.program_id(1)
    nj = pl.num_programs(1)
    half = nj // 2

    @pl.when(j == 0)
    def _init():
        run_min[...] = jnp.full((_BM,), jnp.inf, jnp.float32)
        run_idx[...] = jnp.zeros((_BM,), jnp.int32)

    z2x = z_ref[...] * 2.0
    z2col = z2_ref[...][:, None]
    sub = _BK // 2
    for s in range(2):
        c = c_ref[pl.ds(s * sub, sub), :]
        dot2 = lax.dot_general(z2x, c, (((1,), (1,)), ((), ())),
                               preferred_element_type=jnp.float32)
        d = (z2col + c2_ref[pl.ds(s * sub, sub)][None, :]) - dot2
        m = jnp.min(d, axis=1)
        ii = lax.broadcasted_iota(jnp.int32, d.shape, 1)
        cand = jnp.where(d == m[:, None], ii, jnp.int32(2**31 - 1))
        am = jnp.min(cand, axis=1) + (j * _BK + s * sub)
        better = m < run_min[...]
        run_min[...] = jnp.where(better, m, run_min[...])
        run_idx[...] = jnp.where(better, am, run_idx[...])

    @pl.when(j == half - 1)
    def _snapshot_first_half():
        m0_ref[...] = run_min[...]
        i0_ref[...] = run_idx[...]
        run_min[...] = jnp.full((_BM,), jnp.inf, jnp.float32)
        run_idx[...] = jnp.zeros((_BM,), jnp.int32)

    @pl.when(j == nj - 1)
    def _flush():
        m0b = m0_ref[...].astype(jnp.bfloat16).astype(jnp.float32)
        take1 = run_min[...] < m0b
        idx_ref[...] = jnp.where(take1, run_idx[...], i0_ref[...])


def _compute_indices(z2, c2, z_e, codebook):
    grid = (_B // _BM, _NUM_EMBEDDINGS // _BK)
    return pl.pallas_call(
        _argmin_body,
        grid=grid,
        in_specs=[
            pl.BlockSpec((_BM,), lambda i, j: (i,)),
            pl.BlockSpec((_BK,), lambda i, j: (j,)),
            pl.BlockSpec((_BM, _EMBEDDING_DIM), lambda i, j: (i, 0)),
            pl.BlockSpec((_BK, _EMBEDDING_DIM), lambda i, j: (j, 0)),
        ],
        out_specs=pl.BlockSpec((_BM,), lambda i, j: (i,)),
        out_shape=jax.ShapeDtypeStruct((_B,), jnp.int32),
        scratch_shapes=[
            pltpu.VMEM((_BM,), jnp.float32),
            pltpu.VMEM((_BM,), jnp.int32),
            pltpu.VMEM((_BM,), jnp.float32),
            pltpu.VMEM((_BM,), jnp.int32),
        ],
        compiler_params=pltpu.CompilerParams(
            dimension_semantics=("arbitrary", "arbitrary")),
    )(z2, c2, z_e, codebook)


def _sc_body(cb_hbm, idx_hbm, zq_hbm, hist_hbm, idx_v, rows_v, hist_v, sem):
    wid = lax.axis_index("s") * _NC + lax.axis_index("c")
    base = wid * _BPW
    pltpu.sync_copy(idx_hbm.at[pl.ds(base, _BPW)], idx_v)
    gather = pltpu.async_copy(cb_hbm.at[idx_v], rows_v, sem)

    def zero_body(i, carry):
        hist_v[pl.ds(i * _LANES, _LANES)] = jnp.zeros((_LANES,), jnp.float32)
        return carry

    lax.fori_loop(0, _NUM_EMBEDDINGS // _LANES, zero_body, 0)

    def hist_body(i, carry):
        v = idx_v[pl.ds(i * _LANES, _LANES)]
        cnt, last = plsc.scan_count(v)
        plsc.addupdate_scatter(hist_v, [v], cnt.astype(jnp.float32), mask=last)
        return carry

    lax.fori_loop(0, _BPW // _LANES, hist_body, 0)

    gather.wait()
    pltpu.sync_copy(rows_v, zq_hbm.at[pl.ds(base, _BPW)])
    pltpu.sync_copy(hist_v, hist_hbm.at[wid])


def _sc_gather_hist(codebook_pad, indices):
    mesh = plsc.VectorSubcoreMesh(core_axis_name="c", subcore_axis_name="s")
    run = pl.kernel(
        _sc_body,
        out_type=(
            jax.ShapeDtypeStruct((_B, 128), jnp.float32),
            jax.ShapeDtypeStruct((_NW, _NUM_EMBEDDINGS), jnp.float32),
        ),
        mesh=mesh,
        scratch_types=[
            pltpu.VMEM((_BPW,), jnp.int32),
            pltpu.VMEM((_BPW, 128), jnp.float32),
            pltpu.VMEM((_NUM_EMBEDDINGS,), jnp.float32),
            pltpu.SemaphoreType.DMA,
        ],
        compiler_params=pltpu.CompilerParams(needs_layout_passes=False),
    )
    return run(codebook_pad, indices)


def _epilogue_body(ze_ref, zqp_ref, hist_ref, zq_ref, loss_ref, perp_ref):
    zq = zqp_ref[...][:, : _EMBEDDING_DIM]
    zq_ref[...] = zq
    diff = zq - ze_ref[...]
    l = jnp.sum(diff * diff) * (1.0 / (_B * _EMBEDDING_DIM))
    loss_ref[0, 0] = l + _BETA * l
    counts = jnp.sum(hist_ref[...], axis=0)
    p = counts * (1.0 / _B)
    entropy = -jnp.sum(p * jnp.log(p + 1e-10))
    perp_ref[0, 0] = jnp.exp(entropy)


def _epilogue(z_e, zq_pad, hist):
    return pl.pallas_call(
        _epilogue_body,
        out_specs=[
            pl.BlockSpec((_B, _EMBEDDING_DIM), lambda: (0, 0)),
            pl.BlockSpec(memory_space=pltpu.SMEM),
            pl.BlockSpec(memory_space=pltpu.SMEM),
        ],
        out_shape=[
            jax.ShapeDtypeStruct((_B, _EMBEDDING_DIM), jnp.float32),
            jax.ShapeDtypeStruct((1, 1), jnp.float32),
            jax.ShapeDtypeStruct((1, 1), jnp.float32),
        ],
    )(z_e, zq_pad, hist)


def kernel(z_e, codebook):
    z2 = jnp.sum(z_e ** 2, axis=1)
    c2 = jnp.sum(codebook ** 2, axis=1)
    cb_pad = jnp.pad(codebook, ((0, 0), (0, 128 - _EMBEDDING_DIM)))
    indices = _compute_indices(z2, c2, z_e, codebook)
    zq_pad, hist = _sc_gather_hist(cb_pad, indices)
    z_q, loss, perp = _epilogue(z_e, zq_pad, hist)
    return (z_q, indices, loss[0, 0], perp[0, 0])


# speedup vs baseline: 1.2183x; 1.2183x over previous
import jax
import jax.numpy as jnp
from jax import lax
from jax.experimental import pallas as pl
from jax.experimental.pallas import tpu as pltpu
from jax.experimental.pallas import tpu_sc as plsc

_NUM_EMBEDDINGS = 8192
_EMBEDDING_DIM = 64
_BETA = 0.25
_B = 16384

_NC = 2
_NS = 16
_NW = _NC * _NS
_LANES = 16
_BPW = _B // _NW

_BM = 2048
_BK = 4096


def _argmin_body(z2_ref, c2_ref, z_ref, c_ref, idx_ref,
                 run_min, run_idx, m0_ref, i0_ref):
    j = pl.program_id(1)
    nj = pl.num_programs(1)
    half = nj // 2

    @pl.when(j == 0)
    def _init():
        run_min[...] = jnp.full((_BM,), jnp.inf, jnp.float32)
        run_idx[...] = jnp.zeros((_BM,), jnp.int32)

    z2x = z_ref[...] * 2.0
    c = c_ref[...]
    dot2 = lax.dot_general(z2x, c, (((1,), (1,)), ((), ())),
                           preferred_element_type=jnp.float32)
    d = (z2_ref[...][:, None] + c2_ref[...][None, :]) - dot2
    m = jnp.min(d, axis=1)
    ii = lax.broadcasted_iota(jnp.int32, d.shape, 1)
    cand = jnp.where(d == m[:, None], ii, jnp.int32(2**31 - 1))
    am = jnp.min(cand, axis=1) + j * _BK
    better = m < run_min[...]
    run_min[...] = jnp.where(better, m, run_min[...])
    run_idx[...] = jnp.where(better, am, run_idx[...])

    @pl.when(j == half - 1)
    def _snapshot_first_half():
        m0_ref[...] = run_min[...]
        i0_ref[...] = run_idx[...]
        run_min[...] = jnp.full((_BM,), jnp.inf, jnp.float32)
        run_idx[...] = jnp.zeros((_BM,), jnp.int32)

    @pl.when(j == nj - 1)
    def _flush():
        m0b = m0_ref[...].astype(jnp.bfloat16).astype(jnp.float32)
        take1 = run_min[...] < m0b
        idx_ref[...] = jnp.where(take1, run_idx[...], i0_ref[...])


def _compute_indices(z2, c2, z_e, codebook):
    grid = (_B // _BM, _NUM_EMBEDDINGS // _BK)
    return pl.pallas_call(
        _argmin_body,
        grid=grid,
        in_specs=[
            pl.BlockSpec((_BM,), lambda i, j: (i,)),
            pl.BlockSpec((_BK,), lambda i, j: (j,)),
            pl.BlockSpec((_BM, _EMBEDDING_DIM), lambda i, j: (i, 0)),
            pl.BlockSpec((_BK, _EMBEDDING_DIM), lambda i, j: (j, 0)),
        ],
        out_specs=pl.BlockSpec((_BM,), lambda i, j: (i,)),
        out_shape=jax.ShapeDtypeStruct((_B,), jnp.int32),
        scratch_shapes=[
            pltpu.VMEM((_BM,), jnp.float32),
            pltpu.VMEM((_BM,), jnp.int32),
            pltpu.VMEM((_BM,), jnp.float32),
            pltpu.VMEM((_BM,), jnp.int32),
        ],
        compiler_params=pltpu.CompilerParams(
            dimension_semantics=("arbitrary", "arbitrary")),
    )(z2, c2, z_e, codebook)


def _sc_body(cb_hbm, idx_hbm, zq_hbm, hist_hbm, idx_v, rows_v, hist_v, sem):
    wid = lax.axis_index("s") * _NC + lax.axis_index("c")
    base = wid * _BPW
    pltpu.sync_copy(idx_hbm.at[pl.ds(base, _BPW)], idx_v)
    gather = pltpu.async_copy(cb_hbm.at[idx_v], rows_v, sem)

    def zero_body(i, carry):
        hist_v[pl.ds(i * _LANES, _LANES)] = jnp.zeros((_LANES,), jnp.float32)
        return carry

    lax.fori_loop(0, _NUM_EMBEDDINGS // _LANES, zero_body, 0)

    def hist_body(i, carry):
        v = idx_v[pl.ds(i * _LANES, _LANES)]
        cnt, last = plsc.scan_count(v)
        plsc.addupdate_scatter(hist_v, [v], cnt.astype(jnp.float32), mask=last)
        return carry

    lax.fori_loop(0, _BPW // _LANES, hist_body, 0)

    gather.wait()
    pltpu.sync_copy(rows_v, zq_hbm.at[pl.ds(base, _BPW)])
    pltpu.sync_copy(hist_v, hist_hbm.at[wid])


def _sc_gather_hist(codebook_pad, indices):
    mesh = plsc.VectorSubcoreMesh(core_axis_name="c", subcore_axis_name="s")
    run = pl.kernel(
        _sc_body,
        out_type=(
            jax.ShapeDtypeStruct((_B, 128), jnp.float32),
            jax.ShapeDtypeStruct((_NW, _NUM_EMBEDDINGS), jnp.float32),
        ),
        mesh=mesh,
        scratch_types=[
            pltpu.VMEM((_BPW,), jnp.int32),
            pltpu.VMEM((_BPW, 128), jnp.float32),
            pltpu.VMEM((_NUM_EMBEDDINGS,), jnp.float32),
            pltpu.SemaphoreType.DMA,
        ],
        compiler_params=pltpu.CompilerParams(needs_layout_passes=False),
    )
    return run(codebook_pad, indices)


def _epilogue_body(ze_ref, zqp_ref, hist_ref, zq_ref, loss_ref, perp_ref):
    zq = zqp_ref[...][:, : _EMBEDDING_DIM]
    zq_ref[...] = zq
    diff = zq - ze_ref[...]
    l = jnp.sum(diff * diff) * (1.0 / (_B * _EMBEDDING_DIM))
    loss_ref[0, 0] = l + _BETA * l
    counts = jnp.sum(hist_ref[...], axis=0)
    p = counts * (1.0 / _B)
    entropy = -jnp.sum(p * jnp.log(p + 1e-10))
    perp_ref[0, 0] = jnp.exp(entropy)


def _epilogue(z_e, zq_pad, hist):
    return pl.pallas_call(
        _epilogue_body,
        out_specs=[
            pl.BlockSpec((_B, _EMBEDDING_DIM), lambda: (0, 0)),
            pl.BlockSpec(memory_space=pltpu.SMEM),
            pl.BlockSpec(memory_space=pltpu.SMEM),
        ],
        out_shape=[
            jax.ShapeDtypeStruct((_B, _EMBEDDING_DIM), jnp.float32),
            jax.ShapeDtypeStruct((1, 1), jnp.float32),
            jax.ShapeDtypeStruct((1, 1), jnp.float32),
        ],
    )(z_e, zq_pad, hist)


def kernel(z_e, codebook):
    z2 = jnp.sum(z_e ** 2, axis=1)
    c2 = jnp.sum(codebook ** 2, axis=1)
    cb_pad = jnp.pad(codebook, ((0, 0), (0, 128 - _EMBEDDING_DIM)))
    indices = _compute_indices(z2, c2, z_e, codebook)
    zq_pad, hist = _sc_gather_hist(cb_pad, indices)
    z_q, loss, perp = _epilogue(z_e, zq_pad, hist)
    return (z_q, indices, loss[0, 0], perp[0, 0])
